# baseline (device time: 205326 ns/iter reference)
import numpy as np

import jax
import jax.numpy as jnp
from jax import lax
from jax.experimental import pallas as pl
from jax.experimental.pallas import tpu as pltpu

N_DEV = 32
CW_HOPS = 16
CCW_HOPS = 15

_MESH_COORDS = []
for _z in range(4):
    for _yi, _y in enumerate(range(4)):
        _row = [(0, _y, _z), (1, _y, _z)]
        if _yi % 2:
            _row.reverse()
        _MESH_COORDS.extend(_row)
_COORD_TO_MESH = {c: i for i, c in enumerate(_MESH_COORDS)}

_P = []
for _z in range(4):
    _ys = range(4) if _z % 2 == 0 else range(3, -1, -1)
    _P.extend((_y, _z) for _y in _ys)
_CYCLE_COORDS = [(0, y, z) for (y, z) in _P] + [(1, y, z) for (y, z) in reversed(_P)]
_CYCLE_MESH = [_COORD_TO_MESH[c] for c in _CYCLE_COORDS]
_POS = {m: p for p, m in enumerate(_CYCLE_MESH)}

_META = np.zeros((N_DEV, 2 + CW_HOPS + CCW_HOPS), np.int32)
for _d in range(N_DEV):
    _p = _POS[_d]
    _META[_d, 0] = _CYCLE_MESH[(_p - 1) % N_DEV]
    _META[_d, 1] = _CYCLE_MESH[(_p + 1) % N_DEV]
    for _h in range(1, CW_HOPS + 1):
        _META[_d, 1 + _h] = _CYCLE_MESH[(_p - _h) % N_DEV]
    for _g in range(1, CCW_HOPS + 1):
        _META[_d, 1 + CW_HOPS + _g] = _CYCLE_MESH[(_p + _g) % N_DEV]


def kernel(x, w_mat):
    m_per, k = x.shape
    _, n_per = w_mat.shape
    m_total = N_DEV * m_per

    x = x.astype(jnp.bfloat16)
    w_mat = w_mat.astype(jnp.bfloat16)

    my = lax.axis_index("i")
    meta = jnp.asarray(_META)[my]

    n_sub = 4
    rows_sub = m_per // n_sub

    def body(x_ref, w_ref, meta_ref, out_ref, gather_ref,
             cw_send_sems, cw_recv_sems, ccw_send_sems, ccw_recv_sems):
        left = meta_ref[0]
        right = meta_ref[1]

        barrier_sem = pltpu.get_barrier_semaphore()
        for nbr in (left, right):
            pl.semaphore_signal(
                barrier_sem, inc=1,
                device_id=(nbr,), device_id_type=pl.DeviceIdType.MESH,
            )
        pl.semaphore_wait(barrier_sem, 2)

        def sub(ref, j):
            return ref.at[pl.ds(j * rows_sub, rows_sub)]

        def make_cw(h, j):
            src = x_ref if h == 1 else gather_ref.at[h - 1]
            return pltpu.make_async_remote_copy(
                src_ref=sub(src, j),
                dst_ref=sub(gather_ref.at[h], j),
                send_sem=cw_send_sems.at[h - 1, j],
                recv_sem=cw_recv_sems.at[h - 1, j],
                device_id=(right,),
                device_id_type=pl.DeviceIdType.MESH,
            )

        def make_ccw(g, j):
            src = x_ref if g == 1 else gather_ref.at[33 - g]
            return pltpu.make_async_remote_copy(
                src_ref=sub(src, j),
                dst_ref=sub(gather_ref.at[32 - g], j),
                send_sem=ccw_send_sems.at[g - 1, j],
                recv_sem=ccw_recv_sems.at[g - 1, j],
                device_id=(left,),
                device_id_type=pl.DeviceIdType.MESH,
            )

        def strip(origin, chunk):
            y = jnp.dot(chunk, w_ref[...], preferred_element_type=jnp.float32)
            out_ref[pl.ds(origin * m_per, m_per), :] = jnp.maximum(y, 0.0)

        def batch(lo, hi, origin_col):
            n = hi - lo
            chunk = jnp.reshape(
                gather_ref[pl.ds(lo, n), :, :], (n * m_per, k)
            )
            y = jnp.dot(chunk, w_ref[...], preferred_element_type=jnp.float32)
            y = jnp.maximum(y, 0.0)
            for t in range(n):
                origin = meta_ref[origin_col(lo + t)]
                out_ref[pl.ds(origin * m_per, m_per), :] = (
                    y[t * m_per:(t + 1) * m_per, :]
                )

        cw = {(h, j): make_cw(h, j)
              for h in range(1, CW_HOPS + 1) for j in range(n_sub)}
        ccw = {(g, j): make_ccw(g, j)
               for g in range(1, CCW_HOPS + 1) for j in range(n_sub)}

        for j in range(n_sub):
            cw[1, j].start()
            ccw[1, j].start()
        strip(lax.axis_index("i"), x_ref[...])

        for step in range(1, CW_HOPS + 1):
            for j in range(n_sub):
                cw[step, j].wait_recv()
                if step + 1 <= CW_HOPS:
                    cw[step + 1, j].start()
                if step <= CCW_HOPS:
                    ccw[step, j].wait_recv()
                    if step + 1 <= CCW_HOPS:
                        ccw[step + 1, j].start()
            if step % 4 == 0:
                batch(step - 3, step + 1, lambda s: 1 + s)
                ccw_lo = max(32 - step, 32 - CCW_HOPS)
                batch(ccw_lo, 36 - step, lambda s: 1 + CW_HOPS + (32 - s))

        for h in range(1, CW_HOPS + 1):
            for j in range(n_sub):
                cw[h, j].wait_send()
        for g in range(1, CCW_HOPS + 1):
            for j in range(n_sub):
                ccw[g, j].wait_send()

    return pl.pallas_call(
        body,
        out_shape=jax.ShapeDtypeStruct((m_total, n_per), jnp.float32),
        in_specs=[
            pl.BlockSpec(memory_space=pltpu.VMEM),
            pl.BlockSpec(memory_space=pltpu.VMEM),
            pl.BlockSpec(memory_space=pltpu.SMEM),
        ],
        out_specs=pl.BlockSpec(memory_space=pltpu.VMEM),
        scratch_shapes=[
            pltpu.VMEM((N_DEV, m_per, k), jnp.bfloat16),
            pltpu.SemaphoreType.DMA((CW_HOPS, 4)),
            pltpu.SemaphoreType.DMA((CW_HOPS, 4)),
            pltpu.SemaphoreType.DMA((CCW_HOPS, 4)),
            pltpu.SemaphoreType.DMA((CCW_HOPS, 4)),
        ],
        compiler_params=pltpu.CompilerParams(
            collective_id=0,
            vmem_limit_bytes=100 * 1024 * 1024,
        ),
    )(x, w_mat, meta)


# device time: 199933 ns/iter; 1.0270x vs baseline; 1.0270x over previous
import numpy as np

import jax
import jax.numpy as jnp
from jax import lax
from jax.experimental import pallas as pl
from jax.experimental.pallas import tpu as pltpu

N_DEV = 32
CW_HOPS = 16
CCW_HOPS = 16
N_SUB = 4

_MESH_COORDS = []
for _z in range(4):
    for _yi, _y in enumerate(range(4)):
        _row = [(0, _y, _z), (1, _y, _z)]
        if _yi % 2:
            _row.reverse()
        _MESH_COORDS.extend(_row)
_COORD_TO_MESH = {c: i for i, c in enumerate(_MESH_COORDS)}

_P = []
for _z in range(4):
    _ys = range(4) if _z % 2 == 0 else range(3, -1, -1)
    _P.extend((_y, _z) for _y in _ys)
_CYCLE_COORDS = [(0, y, z) for (y, z) in _P] + [(1, y, z) for (y, z) in reversed(_P)]
_CYCLE_MESH = [_COORD_TO_MESH[c] for c in _CYCLE_COORDS]
_POS = {m: p for p, m in enumerate(_CYCLE_MESH)}

_META = np.zeros((N_DEV, 2 + (N_DEV - 1)), np.int32)
for _d in range(N_DEV):
    _p = _POS[_d]
    _META[_d, 0] = _CYCLE_MESH[(_p - 1) % N_DEV]
    _META[_d, 1] = _CYCLE_MESH[(_p + 1) % N_DEV]
    for _s in range(1, N_DEV):
        _META[_d, 1 + _s] = _CYCLE_MESH[(_p - _s) % N_DEV]


def kernel(x, w_mat):
    m_per, k = x.shape
    _, n_per = w_mat.shape
    m_total = N_DEV * m_per
    rows_sub = m_per // N_SUB

    x = x.astype(jnp.bfloat16)
    w_mat = w_mat.astype(jnp.bfloat16)

    my = lax.axis_index("i")
    meta = jnp.asarray(_META)[my]

    cw_keys = [(h, j) for h in range(1, CW_HOPS + 1) for j in range(N_SUB)
               if h < 16 or j in (0, 1)]
    ccw_keys = [(g, j) for g in range(1, CCW_HOPS + 1) for j in range(N_SUB)
                if g < 16 or j in (2, 3)]

    def body(x_ref, w_ref, meta_ref, out_ref, gather_ref,
             cw_send_sems, cw_recv_sems, ccw_send_sems, ccw_recv_sems):
        left = meta_ref[0]
        right = meta_ref[1]

        barrier_sem = pltpu.get_barrier_semaphore()
        for nbr in (left, right):
            pl.semaphore_signal(
                barrier_sem, inc=1,
                device_id=(nbr,), device_id_type=pl.DeviceIdType.MESH,
            )
        pl.semaphore_wait(barrier_sem, 2)

        def sub_rows(slot, j):
            return pl.ds(slot * m_per + j * rows_sub, rows_sub)

        def make_cw(h, j):
            src = (x_ref.at[pl.ds(j * rows_sub, rows_sub)] if h == 1
                   else gather_ref.at[sub_rows(h - 1, j)])
            return pltpu.make_async_remote_copy(
                src_ref=src,
                dst_ref=gather_ref.at[sub_rows(h, j)],
                send_sem=cw_send_sems.at[h - 1, j],
                recv_sem=cw_recv_sems.at[h - 1, j],
                device_id=(right,),
                device_id_type=pl.DeviceIdType.MESH,
            )

        def make_ccw(g, j):
            src = (x_ref.at[pl.ds(j * rows_sub, rows_sub)] if g == 1
                   else gather_ref.at[sub_rows(33 - g, j)])
            return pltpu.make_async_remote_copy(
                src_ref=src,
                dst_ref=gather_ref.at[sub_rows(32 - g, j)],
                send_sem=ccw_send_sems.at[g - 1, j],
                recv_sem=ccw_recv_sems.at[g - 1, j],
                device_id=(left,),
                device_id_type=pl.DeviceIdType.MESH,
            )

        def store_batch(lo, n, y):
            for t in range(n):
                origin = meta_ref[1 + lo + t]
                out_ref[pl.ds(origin * m_per, m_per), :] = (
                    y[t * m_per:(t + 1) * m_per, :]
                )

        def batch(lo, hi):
            n = hi - lo
            chunk = gather_ref[pl.ds(lo * m_per, n * m_per), :]
            y = jnp.dot(chunk, w_ref[...], preferred_element_type=jnp.float32)
            store_batch(lo, n, jnp.maximum(y, 0.0))

        cw = {key: make_cw(*key) for key in cw_keys}
        ccw = {key: make_ccw(*key) for key in ccw_keys}

        for j in range(N_SUB):
            cw[1, j].start()
            ccw[1, j].start()
        y0 = jnp.dot(x_ref[...], w_ref[...],
                     preferred_element_type=jnp.float32)
        out_ref[pl.ds(lax.axis_index("i") * m_per, m_per), :] = (
            jnp.maximum(y0, 0.0)
        )

        for step in range(1, 17):
            for j in range(N_SUB):
                if (step, j) in cw:
                    cw[step, j].wait_recv()
                if (step + 1, j) in cw:
                    cw[step + 1, j].start()
                if (step, j) in ccw:
                    ccw[step, j].wait_recv()
                if (step + 1, j) in ccw:
                    ccw[step + 1, j].start()
            if step % 4 == 0:
                batch(step - 3, step + 1)
                if step < 16:
                    batch(32 - step, 36 - step)
                else:
                    batch(17, 20)

        for key in cw_keys:
            cw[key].wait_send()
        for key in ccw_keys:
            ccw[key].wait_send()

    return pl.pallas_call(
        body,
        out_shape=jax.ShapeDtypeStruct((m_total, n_per), jnp.float32),
        in_specs=[
            pl.BlockSpec(memory_space=pltpu.VMEM),
            pl.BlockSpec(memory_space=pltpu.VMEM),
            pl.BlockSpec(memory_space=pltpu.SMEM),
        ],
        out_specs=pl.BlockSpec(memory_space=pltpu.VMEM),
        scratch_shapes=[
            pltpu.VMEM((N_DEV * m_per, k), jnp.bfloat16),
            pltpu.SemaphoreType.DMA((CW_HOPS, N_SUB)),
            pltpu.SemaphoreType.DMA((CW_HOPS, N_SUB)),
            pltpu.SemaphoreType.DMA((CCW_HOPS, N_SUB)),
            pltpu.SemaphoreType.DMA((CCW_HOPS, N_SUB)),
        ],
        compiler_params=pltpu.CompilerParams(
            collective_id=0,
            vmem_limit_bytes=100 * 1024 * 1024,
        ),
    )(x, w_mat, meta)
